# baseline (speedup 1.0000x reference)
import jax
import jax.numpy as jnp
from jax import lax
from jax.experimental import pallas as pl
from jax.experimental.pallas import tpu as pltpu
from jax.experimental.pallas import tpu_sc as plsc

N = 262144
TR, XR, YR, D = 600, 300, 300, 2
NC, NS = 2, 16
NW = NC * NS
PW = N // NW
CH = 1024
NCHUNK = PW // CH
GI = 128
NG = CH * 9 // GI
KFL = 8


def _body(t_hbm, x_hbm, y_hbm, emb_hbm, out_hbm,
          t_v, x_v, y_v, idx_v, om_v, o0_v, op_v, blk_v, out_v, sem):
    wid = lax.axis_index("s") * NC + lax.axis_index("c")
    lane = lax.iota(jnp.int32, 16)

    for g in range(NCHUNK):
        p0 = wid * PW + g * CH
        pltpu.sync_copy(t_hbm.at[pl.ds(p0, CH)], t_v)
        pltpu.sync_copy(x_hbm.at[pl.ds(p0, CH)], x_v)
        pltpu.sync_copy(y_hbm.at[pl.ds(p0, CH)], y_v)

        def build_body(it, _):
            s = it * 16
            tv = t_v[pl.ds(s, 16)]
            xv = x_v[pl.ds(s, 16)]
            yv = y_v[pl.ds(s, 16)]
            ti = jnp.clip((tv * 599.0 + 0.5).astype(jnp.int32), 0, TR - 1)
            xi = jnp.clip((xv * 299.0 + 0.5).astype(jnp.int32), 0, XR - 1)
            yi = jnp.clip((yv * 299.0 + 0.5).astype(jnp.int32), 0, YR - 1)
            tb = ti * (XR * YR)
            rows = [tb + jnp.maximum(xi - 1, 0) * YR,
                    tb + xi * YR,
                    tb + jnp.minimum(xi + 1, XR - 1) * YR]
            cols = [jnp.maximum(yi - 1, 0), yi, jnp.minimum(yi + 1, YR - 1)]
            om_v[pl.ds(s, 16)] = (cols[0] & 3) << 1
            o0_v[pl.ds(s, 16)] = (cols[1] & 3) << 1
            op_v[pl.ds(s, 16)] = (cols[2] & 3) << 1
            sbase = (lane + s) * 9
            k = 0
            for r in rows:
                for c in cols:
                    plsc.store_scatter(idx_v, [sbase + k], (r + c) >> 2)
                    k += 1
            return 0

        lax.fori_loop(0, CH // 16, build_body, 0)

        def fire(r):
            pltpu.make_async_copy(
                emb_hbm.at[idx_v.at[pl.ds(r * GI, GI)]],
                blk_v.at[pl.ds(r * GI, GI)],
                sem).start()

        def drain(r):
            pltpu.make_async_copy(
                emb_hbm.at[idx_v.at[pl.ds(r * GI, GI)]],
                blk_v.at[pl.ds(r * GI, GI)],
                sem).wait()

        for r in range(KFL):
            fire(r)

        def gat_body(r, _):
            fire(r + KFL)
            drain(r)
            return 0

        lax.fori_loop(0, NG - KFL, gat_body, 0)

        def drain_body(r, _):
            drain(r)
            return 0

        lax.fori_loop(NG - KFL, NG, drain_body, 0)

        def extract_body(it, _):
            s = it * 16
            pb = (lane + s) * 9
            ob = (lane + s) * 18
            offs = [om_v[pl.ds(s, 16)], o0_v[pl.ds(s, 16)], op_v[pl.ds(s, 16)]]
            offs1 = [o + 1 for o in offs]
            for ix in range(3):
                for j in range(3):
                    k = ix * 3 + j
                    row = pb + k
                    v0 = plsc.load_gather(blk_v, [row, offs[j]])
                    v1 = plsc.load_gather(blk_v, [row, offs1[j]])
                    plsc.store_scatter(out_v, [ob + 2 * k], v0)
                    plsc.store_scatter(out_v, [ob + 2 * k + 1], v1)
            return 0

        lax.fori_loop(0, CH // 16, extract_body, 0)

        pltpu.sync_copy(out_v, out_hbm.at[pl.ds(p0 * 18, CH * 18)])


def kernel(inputs, embeddings):
    t = inputs[:, 0]
    x = inputs[:, 1]
    y = inputs[:, 2]
    emb = embeddings.reshape(TR * XR * YR // 4, 8)
    run = pl.kernel(
        _body,
        out_type=jax.ShapeDtypeStruct((N * 18,), jnp.float32),
        mesh=plsc.VectorSubcoreMesh(core_axis_name="c", subcore_axis_name="s"),
        compiler_params=pltpu.CompilerParams(
            needs_layout_passes=False, use_tc_tiling_on_sc=False),
        scratch_types=[
            pltpu.VMEM((CH,), jnp.float32),
            pltpu.VMEM((CH,), jnp.float32),
            pltpu.VMEM((CH,), jnp.float32),
            pltpu.VMEM((CH * 9,), jnp.int32),
            pltpu.VMEM((CH,), jnp.int32),
            pltpu.VMEM((CH,), jnp.int32),
            pltpu.VMEM((CH,), jnp.int32),
            pltpu.VMEM((CH * 9, 8), jnp.float32),
            pltpu.VMEM((CH * 18,), jnp.float32),
            pltpu.SemaphoreType.DMA,
        ],
    )
    out = run(t, x, y, emb)
    return out.reshape(N, 18)


# speedup vs baseline: 10.9327x; 10.9327x over previous
import jax
import jax.numpy as jnp
from jax import lax
from jax.experimental import pallas as pl
from jax.experimental.pallas import tpu as pltpu
from jax.experimental.pallas import tpu_sc as plsc

N = 262144
TR, XR, YR = 600, 300, 300
NC, NS = 2, 16
NW = NC * NS
PW = N // NW
CH = 512
NCHUNK = PW // CH
GI = 128
NG = CH * 18 // GI
KFL = 8


def _body(t_hbm, x_hbm, y_hbm, emb_hbm, out_hbm,
          t_v, x_v, y_v, idx_v, off_v, blk_v, out_v, sem):
    wid = lax.axis_index("s") * NC + lax.axis_index("c")
    lane = lax.iota(jnp.int32, 16)

    for g in range(NCHUNK):
        p0 = wid * PW + g * CH
        pltpu.sync_copy(t_hbm.at[pl.ds(p0, CH)], t_v)
        pltpu.sync_copy(x_hbm.at[pl.ds(p0, CH)], x_v)
        pltpu.sync_copy(y_hbm.at[pl.ds(p0, CH)], y_v)

        def build_body(it, _):
            s = it * 16
            tv = t_v[pl.ds(s, 16)]
            xv = x_v[pl.ds(s, 16)]
            yv = y_v[pl.ds(s, 16)]
            ti = jnp.clip((tv * 599.0 + 0.5).astype(jnp.int32), 0, TR - 1)
            xi = jnp.clip((xv * 299.0 + 0.5).astype(jnp.int32), 0, XR - 1)
            yi = jnp.clip((yv * 299.0 + 0.5).astype(jnp.int32), 0, YR - 1)
            off_v[pl.ds(s, 16)] = ti & 7
            tqb = ((ti >> 7) << 5) + ((ti & 127) >> 3)
            rows = [jnp.maximum(xi - 1, 0) * YR,
                    xi * YR,
                    jnp.minimum(xi + 1, XR - 1) * YR]
            cols = [jnp.maximum(yi - 1, 0), yi, jnp.minimum(yi + 1, YR - 1)]
            sbase = (lane + s) * 18
            k = 0
            for r in rows:
                for c in cols:
                    b0 = (r + c) * 160 + tqb
                    plsc.store_scatter(idx_v, [sbase + 2 * k], b0)
                    plsc.store_scatter(idx_v, [sbase + 2 * k + 1], b0 + 16)
                    k += 1
            return 0

        lax.fori_loop(0, CH // 16, build_body, 0)

        def fire(r):
            pltpu.make_async_copy(
                emb_hbm.at[idx_v.at[pl.ds(r * GI, GI)]],
                blk_v.at[pl.ds(r * GI, GI)],
                sem).start()

        def drain(r):
            pltpu.make_async_copy(
                emb_hbm.at[idx_v.at[pl.ds(r * GI, GI)]],
                blk_v.at[pl.ds(r * GI, GI)],
                sem).wait()

        for r in range(KFL):
            fire(r)

        def gat_body(r, _):
            fire(r + KFL)
            drain(r)
            return 0

        lax.fori_loop(0, NG - KFL, gat_body, 0)

        def drain_body(r, _):
            drain(r)
            return 0

        lax.fori_loop(NG - KFL, NG, drain_body, 0)

        def extract_body(it, _):
            s = it * 16
            pb = (lane + s) * 18
            ob = (lane + s) * 18
            off = off_v[pl.ds(s, 16)]
            for k in range(9):
                v0 = plsc.load_gather(blk_v, [pb + 2 * k, off])
                v1 = plsc.load_gather(blk_v, [pb + 2 * k + 1, off])
                plsc.store_scatter(out_v, [ob + 2 * k], v0)
                plsc.store_scatter(out_v, [ob + 2 * k + 1], v1)
            return 0

        lax.fori_loop(0, CH // 16, extract_body, 0)

        pltpu.sync_copy(out_v, out_hbm.at[pl.ds(p0 * 18, CH * 18)])


def kernel(inputs, embeddings):
    t = inputs[:, 0]
    x = inputs[:, 1]
    y = inputs[:, 2]
    e = jnp.pad(embeddings, ((0, 40), (0, 0), (0, 0), (0, 0)))
    e = e.reshape(5, 128, XR, YR, 2)
    e = jnp.transpose(e, (2, 3, 0, 4, 1))
    emb = e.reshape(450000 * 32, 8)
    run = pl.kernel(
        _body,
        out_type=jax.ShapeDtypeStruct((N * 18,), jnp.float32),
        mesh=plsc.VectorSubcoreMesh(core_axis_name="c", subcore_axis_name="s"),
        compiler_params=pltpu.CompilerParams(
            needs_layout_passes=False, use_tc_tiling_on_sc=False),
        scratch_types=[
            pltpu.VMEM((CH,), jnp.float32),
            pltpu.VMEM((CH,), jnp.float32),
            pltpu.VMEM((CH,), jnp.float32),
            pltpu.VMEM((CH * 18,), jnp.int32),
            pltpu.VMEM((CH,), jnp.int32),
            pltpu.VMEM((CH * 18, 8), jnp.float32),
            pltpu.VMEM((CH * 18,), jnp.float32),
            pltpu.SemaphoreType.DMA,
        ],
    )
    out = run(t, x, y, emb)
    return out.reshape(N, 18)


# speedup vs baseline: 23.9699x; 2.1925x over previous
import jax
import jax.numpy as jnp
from jax import lax
from jax.experimental import pallas as pl
from jax.experimental.pallas import tpu as pltpu
from jax.experimental.pallas import tpu_sc as plsc

N = 262144
TR, XR, YR = 600, 300, 300
NC, NS = 2, 16
NW = NC * NS
PW = N // NW
CH = 256
NCHUNK = PW // CH
GI = 128
NG = CH * 18 // GI
KFL = 6


def _body(t_hbm, x_hbm, y_hbm, a_hbm, b_hbm, out_hbm,
          t_v, x_v, y_v, idxa_v, idxb_v, off_v, sel_v,
          blka_v, blkb_v, out_v, sem):
    wid = lax.axis_index("s") * NC + lax.axis_index("c")
    lane = lax.iota(jnp.int32, 16)

    def chunk_body(g, _):
        p0 = wid * PW + g * CH
        pltpu.sync_copy(t_hbm.at[pl.ds(p0, CH)], t_v)
        pltpu.sync_copy(x_hbm.at[pl.ds(p0, CH)], x_v)
        pltpu.sync_copy(y_hbm.at[pl.ds(p0, CH)], y_v)

        def build_body(it, _):
            s = it * 16
            tv = t_v[pl.ds(s, 16)]
            xv = x_v[pl.ds(s, 16)]
            yv = y_v[pl.ds(s, 16)]
            ti = jnp.clip((tv * 599.0 + 0.5).astype(jnp.int32), 0, TR - 1)
            xi = jnp.clip((xv * 299.0 + 0.5).astype(jnp.int32), 0, XR - 1)
            yi = jnp.clip((yv * 299.0 + 0.5).astype(jnp.int32), 0, YR - 1)
            off_v[pl.ds(s, 16)] = ti & 7
            sel_v[pl.ds(s, 16)] = (ti < 512).astype(jnp.int32)
            ta = jnp.minimum(ti, 511)
            aa = (ta >> 7) * 2880000 + ((ta & 127) >> 3)
            bb = jnp.maximum(ti - 512, 0) >> 3
            rows = [jnp.maximum(xi - 1, 0) * YR,
                    xi * YR,
                    jnp.minimum(xi + 1, XR - 1) * YR]
            cols = [jnp.maximum(yi - 1, 0), yi, jnp.minimum(yi + 1, YR - 1)]
            sbase = (lane + s) * 18
            k = 0
            for r in rows:
                for c in cols:
                    xy = r + c
                    a0 = xy * 32 + aa
                    b0 = xy * 22 + bb
                    plsc.store_scatter(idxa_v, [sbase + 2 * k], a0)
                    plsc.store_scatter(idxa_v, [sbase + 2 * k + 1], a0 + 16)
                    plsc.store_scatter(idxb_v, [sbase + 2 * k], b0)
                    plsc.store_scatter(idxb_v, [sbase + 2 * k + 1], b0 + 11)
                    k += 1
            return 0

        lax.fori_loop(0, CH // 16, build_body, 0)

        def fire(r):
            pltpu.make_async_copy(
                a_hbm.at[idxa_v.at[pl.ds(r * GI, GI)]],
                blka_v.at[pl.ds(r * GI, GI)], sem).start()
            pltpu.make_async_copy(
                b_hbm.at[idxb_v.at[pl.ds(r * GI, GI)]],
                blkb_v.at[pl.ds(r * GI, GI)], sem).start()

        def drain(r):
            pltpu.make_async_copy(
                a_hbm.at[idxa_v.at[pl.ds(r * GI, GI)]],
                blka_v.at[pl.ds(r * GI, GI)], sem).wait()
            pltpu.make_async_copy(
                b_hbm.at[idxb_v.at[pl.ds(r * GI, GI)]],
                blkb_v.at[pl.ds(r * GI, GI)], sem).wait()

        for r in range(KFL):
            fire(r)

        def gat_body(r, _):
            fire(r + KFL)
            drain(r)
            return 0

        lax.fori_loop(0, NG - KFL, gat_body, 0)

        def drain_body(r, _):
            drain(r)
            return 0

        lax.fori_loop(NG - KFL, NG, drain_body, 0)

        def extract_body(it, _):
            s = it * 16
            pb = (lane + s) * 18
            off = off_v[pl.ds(s, 16)]
            m = sel_v[pl.ds(s, 16)] > 0
            for k in range(18):
                va = plsc.load_gather(blka_v, [pb + k, off])
                vb = plsc.load_gather(blkb_v, [pb + k, off])
                plsc.store_scatter(out_v, [pb + k], jnp.where(m, va, vb))
            return 0

        lax.fori_loop(0, CH // 16, extract_body, 0)

        pltpu.sync_copy(out_v, out_hbm.at[pl.ds(p0 * 18, CH * 18)])
        return 0

    lax.fori_loop(0, NCHUNK, chunk_body, 0)


def kernel(inputs, embeddings):
    t = inputs[:, 0]
    x = inputs[:, 1]
    y = inputs[:, 2]
    a = embeddings[:512].reshape(4, 128, XR, YR, 2)
    a = jnp.transpose(a, (0, 2, 3, 4, 1)).reshape(720000 * 16, 8)
    b = jnp.transpose(embeddings[512:], (1, 2, 3, 0)).reshape(180000 * 11, 8)
    run = pl.kernel(
        _body,
        out_type=jax.ShapeDtypeStruct((N * 18,), jnp.float32),
        mesh=plsc.VectorSubcoreMesh(core_axis_name="c", subcore_axis_name="s"),
        compiler_params=pltpu.CompilerParams(
            needs_layout_passes=False, use_tc_tiling_on_sc=False),
        scratch_types=[
            pltpu.VMEM((CH,), jnp.float32),
            pltpu.VMEM((CH,), jnp.float32),
            pltpu.VMEM((CH,), jnp.float32),
            pltpu.VMEM((CH * 18,), jnp.int32),
            pltpu.VMEM((CH * 18,), jnp.int32),
            pltpu.VMEM((CH,), jnp.int32),
            pltpu.VMEM((CH,), jnp.int32),
            pltpu.VMEM((CH * 18, 8), jnp.float32),
            pltpu.VMEM((CH * 18, 8), jnp.float32),
            pltpu.VMEM((CH * 18,), jnp.float32),
            pltpu.SemaphoreType.DMA,
        ],
    )
    out = run(t, x, y, a, b)
    return out.reshape(N, 18)
